# baseline (speedup 1.0000x reference)
import functools

import jax
import jax.numpy as jnp
from jax import lax
from jax.experimental import pallas as pl
from jax.experimental.pallas import tpu as pltpu
from jax.experimental.pallas import tpu_sc as plsc

_ZERO_RATIO = 0.9
_NC = 2
_NS = 16
_L = 16


def _f32_key(x):
    top = jnp.int32(-2147483648)
    b = lax.bitcast_convert_type(x, jnp.int32)
    return jnp.where(b >= 0, b, (~b) ^ top)


def _sc_body(n, k_rank, rows_per_w, x_hbm, o_hbm, in_v, out_v, cand_v, hist_v):
    chunks = n // _L
    bins = 1024
    low_bits = 22
    low_mask = jnp.int32((1 << low_bits) - 1)
    k = jnp.int32(k_rank)
    lane = lax.iota(jnp.int32, _L)
    lane_hist_base = lane * jnp.int32(bins) + jnp.int32(bins // 2)
    cand_base = lane * jnp.int32(chunks)
    ones = jnp.ones((_L,), jnp.int32)
    zeros = jnp.zeros((_L,), jnp.int32)

    wid = lax.axis_index("s") * _NC + lax.axis_index("c")

    for i in range(rows_per_w):
        row = wid * rows_per_w + i
        pltpu.sync_copy(x_hbm.at[row], in_v)

        @plsc.parallel_loop(0, bins * _L // _L, unroll=8)
        def _zero(c):
            hist_v[pl.ds(c * _L, _L)] = zeros

        @plsc.parallel_loop(0, chunks, unroll=8)
        def _hist(c):
            key = _f32_key(in_v[pl.ds(c * _L, _L)])
            idx = (key >> 22) + lane_hist_base
            plsc.addupdate_scatter(hist_v, [idx], ones)

        @plsc.parallel_loop(0, bins // _L, carry=(jnp.int32(0), zeros, zeros))
        def _scan(c, carry):
            total, nle_acc, cb_acc = carry
            m = hist_v[pl.ds(c * _L, _L)]
            for l in range(1, _L):
                m = m + hist_v[pl.ds(l * bins + c * _L, _L)]
            cum = plsc.cumsum(m) + total
            le = cum <= k
            nle_acc = nle_acc + jnp.where(le, 1, 0)
            cb_acc = jnp.maximum(cb_acc, jnp.where(le, cum, 0))
            return jnp.max(cum), nle_acc, cb_acc

        _, nle_acc, cb_acc = _scan
        b1 = jnp.sum(nle_acc)
        count_before = jnp.max(cb_acc)
        k2 = k - count_before

        @plsc.parallel_loop(0, chunks, carry=cand_base)
        def _compact(c, cursor):
            key = _f32_key(in_v[pl.ds(c * _L, _L)])
            digit = (key >> 22) + jnp.int32(bins // 2)
            hit = digit == b1
            plsc.store_scatter(cand_v, [cursor], key & low_mask, mask=hit)
            return cursor + jnp.where(hit, 1, 0)

        m_l = _compact - cand_base
        max_m = jnp.max(m_l)

        res_low = jnp.int32(0)
        for bit in range(low_bits - 1, -1, -1):
            cand_t = res_low | jnp.int32(1 << bit)

            def _count(j, acc):
                v = plsc.load_gather(cand_v, [cand_base + j])
                ok = (j < m_l) & (v < cand_t)
                return acc + jnp.where(ok, 1, 0)

            cnt = jnp.sum(lax.fori_loop(0, max_m, _count, zeros))
            res_low = jnp.where(cnt <= k2, cand_t, res_low)

        theta_key = ((b1 - jnp.int32(bins // 2)) << low_bits) | res_low

        @plsc.parallel_loop(0, chunks, unroll=8)
        def _mask(c):
            x = in_v[pl.ds(c * _L, _L)]
            key = _f32_key(x)
            out_v[pl.ds(c * _L, _L)] = jnp.where(key > theta_key, x, 0.0)

        pltpu.sync_copy(out_v, o_hbm.at[row])


def kernel(X):
    batch, n = X.shape
    k_rank = int((1.0 - _ZERO_RATIO) * n)
    nw = _NC * _NS
    rows_per_w = batch // nw
    mesh = plsc.VectorSubcoreMesh(core_axis_name="c", subcore_axis_name="s")
    return pl.kernel(
        functools.partial(_sc_body, n, k_rank, rows_per_w),
        out_type=jax.ShapeDtypeStruct((batch, n), jnp.float32),
        mesh=mesh,
        compiler_params=pltpu.CompilerParams(needs_layout_passes=False),
        scratch_types=[
            pltpu.VMEM((n,), jnp.float32),
            pltpu.VMEM((n,), jnp.float32),
            pltpu.VMEM((n,), jnp.int32),
            pltpu.VMEM((1024 * _L,), jnp.int32),
        ],
    )(X)


# speedup vs baseline: 1.0474x; 1.0474x over previous
import functools

import jax
import jax.numpy as jnp
from jax import lax
from jax.experimental import pallas as pl
from jax.experimental.pallas import tpu as pltpu
from jax.experimental.pallas import tpu_sc as plsc

_ZERO_RATIO = 0.9
_NC = 2
_NS = 16
_L = 16


def _f32_key(x):
    top = jnp.int32(-2147483648)
    b = lax.bitcast_convert_type(x, jnp.int32)
    return jnp.where(b >= 0, b, (~b) ^ top)


def _sc_body(n, k_rank, rows_per_w, x_hbm, o_hbm, in_v, out_v, cand_v, hist_v):
    chunks = n // _L
    bins = 1024
    low_bits = 22
    low_mask = jnp.int32((1 << low_bits) - 1)
    k = jnp.int32(k_rank)
    lane = lax.iota(jnp.int32, _L)
    lane_hist_base = lane * jnp.int32(bins) + jnp.int32(bins // 2)
    cand_base = lane * jnp.int32(chunks)
    ones = jnp.ones((_L,), jnp.int32)
    zeros = jnp.zeros((_L,), jnp.int32)

    wid = lax.axis_index("s") * _NC + lax.axis_index("c")

    for i in range(rows_per_w):
        row = wid * rows_per_w + i
        pltpu.sync_copy(x_hbm.at[row], in_v)

        @plsc.parallel_loop(0, bins * _L // _L, unroll=8)
        def _zero(c):
            hist_v[pl.ds(c * _L, _L)] = zeros

        @plsc.parallel_loop(0, chunks, unroll=8)
        def _hist(c):
            key = _f32_key(in_v[pl.ds(c * _L, _L)])
            idx = (key >> 22) + lane_hist_base
            plsc.addupdate_scatter(hist_v, [idx], ones)

        @plsc.parallel_loop(0, bins // _L, unroll=4,
                            carry=(jnp.int32(0), zeros, zeros))
        def _scan(c, carry):
            total, nle_acc, cb_acc = carry
            m = hist_v[pl.ds(c * _L, _L)]
            for l in range(1, _L):
                m = m + hist_v[pl.ds(l * bins + c * _L, _L)]
            cum = plsc.cumsum(m) + total
            le = cum <= k
            nle_acc = nle_acc + jnp.where(le, 1, 0)
            cb_acc = jnp.maximum(cb_acc, jnp.where(le, cum, 0))
            return jnp.max(cum), nle_acc, cb_acc

        _, nle_acc, cb_acc = _scan
        b1 = jnp.sum(nle_acc)
        count_before = jnp.max(cb_acc)
        k2 = k - count_before

        @plsc.parallel_loop(0, chunks, unroll=8, carry=cand_base)
        def _compact(c, cursor):
            key = _f32_key(in_v[pl.ds(c * _L, _L)])
            digit = (key >> 22) + jnp.int32(bins // 2)
            hit = digit == b1
            plsc.store_scatter(cand_v, [cursor], key & low_mask, mask=hit)
            return cursor + jnp.where(hit, 1, 0)

        m_l = _compact - cand_base
        max_m = jnp.max(m_l)

        res_low = jnp.int32(0)
        for bit in range(low_bits - 1, -1, -1):
            cand_t = res_low | jnp.int32(1 << bit)

            def _count(j, acc):
                v = plsc.load_gather(cand_v, [cand_base + j])
                ok = (j < m_l) & (v < cand_t)
                return acc + jnp.where(ok, 1, 0)

            cnt = jnp.sum(lax.fori_loop(0, max_m, _count, zeros))
            res_low = jnp.where(cnt <= k2, cand_t, res_low)

        theta_key = ((b1 - jnp.int32(bins // 2)) << low_bits) | res_low

        @plsc.parallel_loop(0, chunks, unroll=8)
        def _mask(c):
            x = in_v[pl.ds(c * _L, _L)]
            key = _f32_key(x)
            out_v[pl.ds(c * _L, _L)] = jnp.where(key > theta_key, x, 0.0)

        pltpu.sync_copy(out_v, o_hbm.at[row])


def kernel(X):
    batch, n = X.shape
    k_rank = int((1.0 - _ZERO_RATIO) * n)
    nw = _NC * _NS
    rows_per_w = batch // nw
    mesh = plsc.VectorSubcoreMesh(core_axis_name="c", subcore_axis_name="s")
    return pl.kernel(
        functools.partial(_sc_body, n, k_rank, rows_per_w),
        out_type=jax.ShapeDtypeStruct((batch, n), jnp.float32),
        mesh=mesh,
        compiler_params=pltpu.CompilerParams(needs_layout_passes=False),
        scratch_types=[
            pltpu.VMEM((n,), jnp.float32),
            pltpu.VMEM((n,), jnp.float32),
            pltpu.VMEM((n,), jnp.int32),
            pltpu.VMEM((1024 * _L,), jnp.int32),
        ],
    )(X)


# speedup vs baseline: 1.9823x; 1.8926x over previous
import functools

import jax
import jax.numpy as jnp
from jax import lax
from jax.experimental import pallas as pl
from jax.experimental.pallas import tpu as pltpu
from jax.experimental.pallas import tpu_sc as plsc

_ZERO_RATIO = 0.9
_NC = 2
_NS = 16
_L = 16


def _f32_key(x):
    top = jnp.int32(-2147483648)
    b = lax.bitcast_convert_type(x, jnp.int32)
    return jnp.where(b >= 0, b, (~b) ^ top)


def _sc_body(n, k_rank, rows_per_w, x_hbm, o_hbm, in_v, out_v, cand_v, hist_v):
    chunks = n // _L
    bins = 1024
    hist_stride = bins + 1
    cand_stride = chunks + 1
    low_bits = 22
    low_mask = jnp.int32((1 << low_bits) - 1)
    k = jnp.int32(k_rank)
    lane = lax.iota(jnp.int32, _L)
    lane_hist_base = lane * jnp.int32(hist_stride) + jnp.int32(bins // 2)
    cand_base = lane * jnp.int32(cand_stride)
    ones = jnp.ones((_L,), jnp.int32)
    zeros = jnp.zeros((_L,), jnp.int32)

    wid = lax.axis_index("s") * _NC + lax.axis_index("c")

    for i in range(rows_per_w):
        row = wid * rows_per_w + i
        pltpu.sync_copy(x_hbm.at[row], in_v)

        @plsc.parallel_loop(0, hist_stride * _L // _L, unroll=8)
        def _zero(c):
            hist_v[pl.ds(c * _L, _L)] = zeros

        @plsc.parallel_loop(0, chunks, unroll=8)
        def _hist(c):
            key = _f32_key(in_v[pl.ds(c * _L, _L)])
            idx = (key >> 22) + lane_hist_base
            plsc.addupdate_scatter(hist_v, [idx], ones)

        @plsc.parallel_loop(0, bins // _L, unroll=4,
                            carry=(jnp.int32(0), zeros, zeros))
        def _scan(c, carry):
            total, nle_acc, cb_acc = carry
            m = hist_v[pl.ds(c * _L, _L)]
            for l in range(1, _L):
                m = m + hist_v[pl.ds(l * hist_stride + c * _L, _L)]
            cum = plsc.cumsum(m) + total
            le = cum <= k
            nle_acc = nle_acc + jnp.where(le, 1, 0)
            cb_acc = jnp.maximum(cb_acc, jnp.where(le, cum, 0))
            return jnp.max(cum), nle_acc, cb_acc

        _, nle_acc, cb_acc = _scan
        b1 = jnp.sum(nle_acc)
        count_before = jnp.max(cb_acc)
        k2 = k - count_before

        @plsc.parallel_loop(0, chunks, unroll=8, carry=cand_base)
        def _compact(c, cursor):
            key = _f32_key(in_v[pl.ds(c * _L, _L)])
            digit = (key >> 22) + jnp.int32(bins // 2)
            hit = digit == b1
            plsc.store_scatter(cand_v, [cursor], key & low_mask, mask=hit)
            return cursor + jnp.where(hit, 1, 0)

        m_l = _compact - cand_base
        max_m = jnp.max(m_l)

        res_low = jnp.int32(0)
        for bit in range(low_bits - 1, -1, -1):
            cand_t = res_low | jnp.int32(1 << bit)

            def _count(j, acc):
                v = plsc.load_gather(cand_v, [cand_base + j])
                ok = (j < m_l) & (v < cand_t)
                return acc + jnp.where(ok, 1, 0)

            cnt = jnp.sum(lax.fori_loop(0, max_m, _count, zeros))
            res_low = jnp.where(cnt <= k2, cand_t, res_low)

        theta_key = ((b1 - jnp.int32(bins // 2)) << low_bits) | res_low

        @plsc.parallel_loop(0, chunks, unroll=8)
        def _mask(c):
            x = in_v[pl.ds(c * _L, _L)]
            key = _f32_key(x)
            out_v[pl.ds(c * _L, _L)] = jnp.where(key > theta_key, x, 0.0)

        pltpu.sync_copy(out_v, o_hbm.at[row])


def kernel(X):
    batch, n = X.shape
    k_rank = int((1.0 - _ZERO_RATIO) * n)
    nw = _NC * _NS
    rows_per_w = batch // nw
    mesh = plsc.VectorSubcoreMesh(core_axis_name="c", subcore_axis_name="s")
    return pl.kernel(
        functools.partial(_sc_body, n, k_rank, rows_per_w),
        out_type=jax.ShapeDtypeStruct((batch, n), jnp.float32),
        mesh=mesh,
        compiler_params=pltpu.CompilerParams(needs_layout_passes=False),
        scratch_types=[
            pltpu.VMEM((n,), jnp.float32),
            pltpu.VMEM((n,), jnp.float32),
            pltpu.VMEM(((n // _L + 1) * _L,), jnp.int32),
            pltpu.VMEM((1025 * _L,), jnp.int32),
        ],
    )(X)


# speedup vs baseline: 2.1396x; 1.0794x over previous
import functools

import jax
import jax.numpy as jnp
from jax import lax
from jax.experimental import pallas as pl
from jax.experimental.pallas import tpu as pltpu
from jax.experimental.pallas import tpu_sc as plsc

_ZERO_RATIO = 0.9
_NC = 2
_NS = 16
_L = 16


def _sc_body(n, k_rank, rows_per_w, x_hbm, o_hbm, buf_a, buf_b, cand_v, hist_v,
             sin_a, sin_b, sout_a, sout_b):
    chunks = n // _L
    bins = 1024
    hist_stride = bins + 1
    cand_stride = chunks + 1
    low_bits = 22
    low_mask = jnp.int32((1 << low_bits) - 1)
    top = jnp.int32(-2147483648)
    k = jnp.int32(k_rank)
    lane = lax.iota(jnp.int32, _L)
    lane_hist_base = lane * jnp.int32(hist_stride)
    cand_base = lane * jnp.int32(cand_stride)
    ones = jnp.ones((_L,), jnp.int32)
    zeros = jnp.zeros((_L,), jnp.int32)

    bufs = (buf_a, buf_b)
    sins = (sin_a, sin_b)
    souts = (sout_a, sout_b)

    wid = lax.axis_index("s") * _NC + lax.axis_index("c")
    base_row = wid * rows_per_w

    in_dma = [None] * rows_per_w
    out_dma = [None] * rows_per_w
    in_dma[0] = pltpu.async_copy(x_hbm.at[base_row], bufs[0], sins[0])
    if rows_per_w > 1:
        in_dma[1] = pltpu.async_copy(x_hbm.at[base_row + 1], bufs[1], sins[1])

    for i in range(rows_per_w):
        b = i % 2
        in_v = bufs[b]
        in_dma[i].wait()

        @plsc.parallel_loop(0, hist_stride * _L // _L, unroll=8)
        def _zero(c):
            hist_v[pl.ds(c * _L, _L)] = zeros

        @plsc.parallel_loop(0, chunks, unroll=8)
        def _hist(c):
            bb = lax.bitcast_convert_type(in_v[pl.ds(c * _L, _L)], jnp.int32)
            t = bb >> 22
            digit = t ^ jnp.where(t < 0, -1, 512)
            plsc.addupdate_scatter(hist_v, [digit + lane_hist_base], ones)

        if 1 <= i < rows_per_w - 1:
            out_dma[i - 1].wait()
            in_dma[i + 1] = pltpu.async_copy(
                x_hbm.at[base_row + i + 1], bufs[(i + 1) % 2], sins[(i + 1) % 2])

        @plsc.parallel_loop(0, bins // _L, unroll=4,
                            carry=(jnp.int32(0), zeros, zeros))
        def _scan(c, carry):
            total, nle_acc, cb_acc = carry
            m = hist_v[pl.ds(c * _L, _L)]
            for l in range(1, _L):
                m = m + hist_v[pl.ds(l * hist_stride + c * _L, _L)]
            cum = plsc.cumsum(m) + total
            le = cum <= k
            nle_acc = nle_acc + jnp.where(le, 1, 0)
            cb_acc = jnp.maximum(cb_acc, jnp.where(le, cum, 0))
            return jnp.max(cum), nle_acc, cb_acc

        _, nle_acc, cb_acc = _scan
        b1 = jnp.sum(nle_acc)
        count_before = jnp.max(cb_acc)
        k2 = k - count_before

        @plsc.parallel_loop(0, chunks, unroll=8, carry=cand_base)
        def _compact(c, cursor):
            bb = lax.bitcast_convert_type(in_v[pl.ds(c * _L, _L)], jnp.int32)
            t = bb >> 22
            neg = t < 0
            digit = t ^ jnp.where(neg, -1, 512)
            hit = digit == b1
            low = jnp.where(neg, ~bb, bb) & low_mask
            plsc.store_scatter(cand_v, [cursor], low, mask=hit)
            return cursor + jnp.where(hit, 1, 0)

        m_l = _compact - cand_base
        max_m = jnp.max(m_l)

        res_low = jnp.int32(0)
        for bit in range(low_bits - 1, -1, -1):
            cand_t = res_low | jnp.int32(1 << bit)

            def _count(j, acc):
                v = plsc.load_gather(cand_v, [cand_base + j])
                ok = (j < m_l) & (v < cand_t)
                return acc + jnp.where(ok, 1, 0)

            cnt = jnp.sum(lax.fori_loop(0, max_m, _count, zeros))
            res_low = jnp.where(cnt <= k2, cand_t, res_low)

        theta_key = ((b1 - jnp.int32(bins // 2)) << low_bits) | res_low
        theta_bits = jnp.where(theta_key >= 0, theta_key, ~(theta_key ^ top))
        theta = lax.bitcast_convert_type(
            jnp.broadcast_to(theta_bits, (_L,)), jnp.float32)

        @plsc.parallel_loop(0, chunks, unroll=8)
        def _mask(c):
            x = in_v[pl.ds(c * _L, _L)]
            in_v[pl.ds(c * _L, _L)] = jnp.where(x > theta, x, 0.0)

        out_dma[i] = pltpu.async_copy(in_v, o_hbm.at[base_row + i], souts[b])

    for i in range(max(0, rows_per_w - 2), rows_per_w):
        out_dma[i].wait()


def kernel(X):
    batch, n = X.shape
    k_rank = int((1.0 - _ZERO_RATIO) * n)
    nw = _NC * _NS
    rows_per_w = batch // nw
    mesh = plsc.VectorSubcoreMesh(core_axis_name="c", subcore_axis_name="s")
    return pl.kernel(
        functools.partial(_sc_body, n, k_rank, rows_per_w),
        out_type=jax.ShapeDtypeStruct((batch, n), jnp.float32),
        mesh=mesh,
        compiler_params=pltpu.CompilerParams(needs_layout_passes=False),
        scratch_types=[
            pltpu.VMEM((n,), jnp.float32),
            pltpu.VMEM((n,), jnp.float32),
            pltpu.VMEM(((n // _L + 1) * _L,), jnp.int32),
            pltpu.VMEM((1025 * _L,), jnp.int32),
            pltpu.SemaphoreType.DMA,
            pltpu.SemaphoreType.DMA,
            pltpu.SemaphoreType.DMA,
            pltpu.SemaphoreType.DMA,
        ],
    )(X)
